# initial kernel scaffold (unmeasured)
import jax
import jax.numpy as jnp
from jax import lax
from jax.experimental import pallas as pl
from jax.experimental.pallas import tpu as pltpu

B, SQ, H, D = 8, 8, 16, 128
SCALE = D ** -0.5


def _flash_partial(Q, K, V):
    skv = K.shape[1]

    def body(q_ref, k_ref, v_ref, num_ref, l_ref):
        q = q_ref[...]
        k = k_ref[...]
        v = v_ref[...]
        s = lax.dot_general(
            q, k, (((1,), (1,)), ((), ())),
            preferred_element_type=jnp.float32,
        ) * SCALE
        p = jnp.exp(s)
        l_ref[...] = jnp.sum(p, axis=1, keepdims=True)
        num_ref[...] = lax.dot_general(
            p, v, (((1,), (0,)), ((), ())),
            preferred_element_type=jnp.float32,
        )

    return pl.pallas_call(
        body,
        grid=(B, H),
        in_specs=[
            pl.BlockSpec((None, SQ, None, D), lambda b, h: (b, 0, h, 0)),
            pl.BlockSpec((None, skv, None, D), lambda b, h: (b, 0, h, 0)),
            pl.BlockSpec((None, skv, None, D), lambda b, h: (b, 0, h, 0)),
        ],
        out_specs=[
            pl.BlockSpec((None, SQ, None, D), lambda b, h: (b, 0, h, 0)),
            pl.BlockSpec((None, SQ, None, 1), lambda b, h: (b, 0, h, 0)),
        ],
        out_shape=[
            jax.ShapeDtypeStruct((B, SQ, H, D), jnp.float32),
            jax.ShapeDtypeStruct((B, SQ, H, 1), jnp.float32),
        ],
    )(Q, K, V)


def _allreduce_combine(num, l):

    def body(num_ref, l_ref, out_ref, rnum_ref, rl_ref,
             nsend, nrecv, lsend, lrecv):
        my_x = lax.axis_index("x")
        my_y = lax.axis_index("y")
        my_z = lax.axis_index("z")
        peer = (1 - my_x, my_y, my_z)

        barrier = pltpu.get_barrier_semaphore()
        pl.semaphore_signal(
            barrier, inc=1, device_id=peer,
            device_id_type=pl.DeviceIdType.MESH,
        )
        pl.semaphore_wait(barrier, 1)

        rdma_n = pltpu.make_async_remote_copy(
            src_ref=num_ref, dst_ref=rnum_ref,
            send_sem=nsend, recv_sem=nrecv,
            device_id=peer, device_id_type=pl.DeviceIdType.MESH,
        )
        rdma_l = pltpu.make_async_remote_copy(
            src_ref=l_ref, dst_ref=rl_ref,
            send_sem=lsend, recv_sem=lrecv,
            device_id=peer, device_id_type=pl.DeviceIdType.MESH,
        )
        rdma_n.start()
        rdma_l.start()
        rdma_n.wait()
        rdma_l.wait()

        out_ref[...] = (num_ref[...] + rnum_ref[...]) / (
            l_ref[...] + rl_ref[...]
        )

    return pl.pallas_call(
        body,
        in_specs=[
            pl.BlockSpec(memory_space=pltpu.VMEM),
            pl.BlockSpec(memory_space=pltpu.VMEM),
        ],
        out_specs=pl.BlockSpec(memory_space=pltpu.VMEM),
        out_shape=jax.ShapeDtypeStruct((B, SQ, H, D), jnp.float32),
        scratch_shapes=[
            pltpu.VMEM((B, SQ, H, D), jnp.float32),
            pltpu.VMEM((B, SQ, H, 1), jnp.float32),
            pltpu.SemaphoreType.DMA,
            pltpu.SemaphoreType.DMA,
            pltpu.SemaphoreType.DMA,
            pltpu.SemaphoreType.DMA,
        ],
        compiler_params=pltpu.CompilerParams(collective_id=0),
    )(num, l)


def kernel(Q, K, V):
    num, l = _flash_partial(Q, K, V)
    return _allreduce_combine(num, l)


# baseline (device time: 273414 ns/iter reference)
import jax
import jax.numpy as jnp
from jax import lax
from jax.experimental import pallas as pl
from jax.experimental.pallas import tpu as pltpu

B, SQ, H, D = 8, 8, 16, 128
SCALE = D ** -0.5


def _flash_partial(Q3, K3, V3, skv):

    def body(q_ref, k_ref, v_ref, num_ref, l_ref):
        q = q_ref[...]
        k = k_ref[...]
        v = v_ref[...]
        s = lax.dot_general(
            q, k, (((1,), (1,)), ((), ())),
            preferred_element_type=jnp.float32,
        ) * SCALE
        p = jnp.exp(s)
        l_ref[...] = jnp.sum(p, axis=1, keepdims=True)
        num_ref[...] = lax.dot_general(
            p, v, (((1,), (0,)), ((), ())),
            preferred_element_type=jnp.float32,
        )

    return pl.pallas_call(
        body,
        grid=(B, H),
        in_specs=[
            pl.BlockSpec((None, SQ, D), lambda b, h: (b, 0, h)),
            pl.BlockSpec((None, skv, D), lambda b, h: (b, 0, h)),
            pl.BlockSpec((None, skv, D), lambda b, h: (b, 0, h)),
        ],
        out_specs=[
            pl.BlockSpec((None, SQ, D), lambda b, h: (b, 0, h)),
            pl.BlockSpec((None, None, SQ, 1), lambda b, h: (b, h, 0, 0)),
        ],
        out_shape=[
            jax.ShapeDtypeStruct((B, SQ, H * D), jnp.float32),
            jax.ShapeDtypeStruct((B, H, SQ, 1), jnp.float32),
        ],
    )(Q3, K3, V3)


def _exchange_sum(num, l):

    def body(num_ref, l_ref, nsum_ref, lsum_ref, rnum_ref, rl_ref,
             nsend, nrecv, lsend, lrecv):
        my_x = lax.axis_index("x")
        my_y = lax.axis_index("y")
        my_z = lax.axis_index("z")
        peer = (1 - my_x, my_y, my_z)

        barrier = pltpu.get_barrier_semaphore()
        pl.semaphore_signal(
            barrier, inc=1, device_id=peer,
            device_id_type=pl.DeviceIdType.MESH,
        )
        pl.semaphore_wait(barrier, 1)

        rdma_n = pltpu.make_async_remote_copy(
            src_ref=num_ref, dst_ref=rnum_ref,
            send_sem=nsend, recv_sem=nrecv,
            device_id=peer, device_id_type=pl.DeviceIdType.MESH,
        )
        rdma_l = pltpu.make_async_remote_copy(
            src_ref=l_ref, dst_ref=rl_ref,
            send_sem=lsend, recv_sem=lrecv,
            device_id=peer, device_id_type=pl.DeviceIdType.MESH,
        )
        rdma_n.start()
        rdma_l.start()
        rdma_n.wait()
        rdma_l.wait()

        nsum_ref[...] = num_ref[...] + rnum_ref[...]
        lsum_ref[...] = l_ref[...] + rl_ref[...]

    return pl.pallas_call(
        body,
        in_specs=[
            pl.BlockSpec(memory_space=pltpu.VMEM),
            pl.BlockSpec(memory_space=pltpu.VMEM),
        ],
        out_specs=[
            pl.BlockSpec(memory_space=pltpu.VMEM),
            pl.BlockSpec(memory_space=pltpu.VMEM),
        ],
        out_shape=[
            jax.ShapeDtypeStruct((B, SQ, H * D), jnp.float32),
            jax.ShapeDtypeStruct((B, H, SQ, 1), jnp.float32),
        ],
        scratch_shapes=[
            pltpu.VMEM((B, SQ, H * D), jnp.float32),
            pltpu.VMEM((B, H, SQ, 1), jnp.float32),
            pltpu.SemaphoreType.DMA,
            pltpu.SemaphoreType.DMA,
            pltpu.SemaphoreType.DMA,
            pltpu.SemaphoreType.DMA,
        ],
        compiler_params=pltpu.CompilerParams(collective_id=0),
    )(num, l)


def _divide(nsum, lsum):

    def body(n_ref, l_ref, out_ref):
        out_ref[...] = n_ref[...] / l_ref[...]

    return pl.pallas_call(
        body,
        grid=(B, H),
        in_specs=[
            pl.BlockSpec((None, SQ, D), lambda b, h: (b, 0, h)),
            pl.BlockSpec((None, None, SQ, 1), lambda b, h: (b, h, 0, 0)),
        ],
        out_specs=pl.BlockSpec((None, SQ, D), lambda b, h: (b, 0, h)),
        out_shape=jax.ShapeDtypeStruct((B, SQ, H * D), jnp.float32),
    )(nsum, lsum)


def kernel(Q, K, V):
    skv = K.shape[1]
    Q3 = Q.reshape(B, SQ, H * D)
    K3 = K.reshape(B, skv, H * D)
    V3 = V.reshape(B, skv, H * D)
    num, l = _flash_partial(Q3, K3, V3, skv)
    nsum, lsum = _exchange_sum(num, l)
    out = _divide(nsum, lsum)
    return out.reshape(B, SQ, H, D)


# device time: 252277 ns/iter; 1.0838x vs baseline; 1.0838x over previous
import jax
import jax.numpy as jnp
from jax import lax
from jax.experimental import pallas as pl
from jax.experimental.pallas import tpu as pltpu

B, SQ, H, D = 8, 8, 16, 128
SCALE = D ** -0.5


def _flash_partial(Q3, K3, V3, skv):
    KC = 256
    nkc = skv // KC

    def body(q_ref, k_ref, v_ref, num_ref, l_ref):
        kc = pl.program_id(1)

        @pl.when(kc == 0)
        def _():
            num_ref[...] = jnp.zeros_like(num_ref)
            l_ref[...] = jnp.zeros_like(l_ref)

        for h in range(H):
            sl = slice(h * D, (h + 1) * D)
            q = q_ref[:, sl]
            k = k_ref[:, sl]
            v = v_ref[:, sl]
            s = lax.dot_general(
                q, k, (((1,), (1,)), ((), ())),
                preferred_element_type=jnp.float32,
            ) * SCALE
            p = jnp.exp(s)
            l_ref[h] += jnp.sum(p, axis=1, keepdims=True)
            num_ref[:, sl] += lax.dot_general(
                p, v, (((1,), (0,)), ((), ())),
                preferred_element_type=jnp.float32,
            )

    return pl.pallas_call(
        body,
        grid=(B, nkc),
        in_specs=[
            pl.BlockSpec((None, SQ, H * D), lambda b, kc: (b, 0, 0)),
            pl.BlockSpec((None, KC, H * D), lambda b, kc: (b, kc, 0)),
            pl.BlockSpec((None, KC, H * D), lambda b, kc: (b, kc, 0)),
        ],
        out_specs=[
            pl.BlockSpec((None, SQ, H * D), lambda b, kc: (b, 0, 0)),
            pl.BlockSpec((None, H, SQ, 1), lambda b, kc: (b, 0, 0, 0)),
        ],
        out_shape=[
            jax.ShapeDtypeStruct((B, SQ, H * D), jnp.float32),
            jax.ShapeDtypeStruct((B, H, SQ, 1), jnp.float32),
        ],
    )(Q3, K3, V3)


def _exchange_sum(num, l):

    def body(num_ref, l_ref, nsum_ref, lsum_ref, rnum_ref, rl_ref,
             nsend, nrecv, lsend, lrecv):
        my_x = lax.axis_index("x")
        my_y = lax.axis_index("y")
        my_z = lax.axis_index("z")
        peer = (1 - my_x, my_y, my_z)

        barrier = pltpu.get_barrier_semaphore()
        pl.semaphore_signal(
            barrier, inc=1, device_id=peer,
            device_id_type=pl.DeviceIdType.MESH,
        )
        pl.semaphore_wait(barrier, 1)

        rdma_n = pltpu.make_async_remote_copy(
            src_ref=num_ref, dst_ref=rnum_ref,
            send_sem=nsend, recv_sem=nrecv,
            device_id=peer, device_id_type=pl.DeviceIdType.MESH,
        )
        rdma_l = pltpu.make_async_remote_copy(
            src_ref=l_ref, dst_ref=rl_ref,
            send_sem=lsend, recv_sem=lrecv,
            device_id=peer, device_id_type=pl.DeviceIdType.MESH,
        )
        rdma_n.start()
        rdma_l.start()
        rdma_n.wait()
        rdma_l.wait()

        nsum_ref[...] = num_ref[...] + rnum_ref[...]
        lsum_ref[...] = l_ref[...] + rl_ref[...]

    return pl.pallas_call(
        body,
        in_specs=[
            pl.BlockSpec(memory_space=pltpu.VMEM),
            pl.BlockSpec(memory_space=pltpu.VMEM),
        ],
        out_specs=[
            pl.BlockSpec(memory_space=pltpu.VMEM),
            pl.BlockSpec(memory_space=pltpu.VMEM),
        ],
        out_shape=[
            jax.ShapeDtypeStruct((B, SQ, H * D), jnp.float32),
            jax.ShapeDtypeStruct((B, H, SQ, 1), jnp.float32),
        ],
        scratch_shapes=[
            pltpu.VMEM((B, SQ, H * D), jnp.float32),
            pltpu.VMEM((B, H, SQ, 1), jnp.float32),
            pltpu.SemaphoreType.DMA,
            pltpu.SemaphoreType.DMA,
            pltpu.SemaphoreType.DMA,
            pltpu.SemaphoreType.DMA,
        ],
        compiler_params=pltpu.CompilerParams(collective_id=0),
    )(num, l)


def _divide(nsum, lsum):

    def body(n_ref, l_ref, out_ref):
        out_ref[...] = n_ref[...] / l_ref[...]

    return pl.pallas_call(
        body,
        grid=(B, H),
        in_specs=[
            pl.BlockSpec((None, SQ, D), lambda b, h: (b, 0, h)),
            pl.BlockSpec((None, None, SQ, 1), lambda b, h: (b, h, 0, 0)),
        ],
        out_specs=pl.BlockSpec((None, SQ, D), lambda b, h: (b, 0, h)),
        out_shape=jax.ShapeDtypeStruct((B, SQ, H * D), jnp.float32),
    )(nsum, lsum)


def kernel(Q, K, V):
    skv = K.shape[1]
    Q3 = Q.reshape(B, SQ, H * D)
    K3 = K.reshape(B, skv, H * D)
    V3 = V.reshape(B, skv, H * D)
    num, l = _flash_partial(Q3, K3, V3, skv)
    nsum, lsum = _exchange_sum(num, l)
    out = _divide(nsum, lsum)
    return out.reshape(B, SQ, H, D)


# device time: 208528 ns/iter; 1.3112x vs baseline; 1.2098x over previous
import jax
import jax.numpy as jnp
from jax import lax
from jax.experimental import pallas as pl
from jax.experimental.pallas import tpu as pltpu

B, SQ, H, D = 8, 8, 16, 128
SCALE = D ** -0.5


def _flash_partial(Q3, K3, V3, skv):
    KC = 512
    nkc = skv // KC

    def body(q_ref, k_ref, v_ref, num_ref, l_ref, p_ref):
        kc = pl.program_id(1)

        @pl.when(kc == 0)
        def _():
            num_ref[...] = jnp.zeros_like(num_ref)
            l_ref[...] = jnp.zeros_like(l_ref)

        for h in range(H):
            sl = slice(h * D, (h + 1) * D)
            psl = slice(h * KC, (h + 1) * KC)
            p_ref[:, psl] = lax.dot_general(
                q_ref[:, sl], k_ref[:, sl], (((1,), (1,)), ((), ())),
                preferred_element_type=jnp.float32,
            )
        p_ref[...] = jnp.exp(p_ref[...] * SCALE)
        for h in range(H):
            sl = slice(h * D, (h + 1) * D)
            psl = slice(h * KC, (h + 1) * KC)
            p = p_ref[:, psl]
            l_ref[h] += jnp.sum(p, axis=1, keepdims=True)
            num_ref[:, sl] += lax.dot_general(
                p, v_ref[:, sl], (((1,), (0,)), ((), ())),
                preferred_element_type=jnp.float32,
            )

    return pl.pallas_call(
        body,
        grid=(B, nkc),
        in_specs=[
            pl.BlockSpec((None, SQ, H * D), lambda b, kc: (b, 0, 0)),
            pl.BlockSpec((None, KC, H * D), lambda b, kc: (b, kc, 0)),
            pl.BlockSpec((None, KC, H * D), lambda b, kc: (b, kc, 0)),
        ],
        out_specs=[
            pl.BlockSpec((None, SQ, H * D), lambda b, kc: (b, 0, 0)),
            pl.BlockSpec((None, H, SQ, 1), lambda b, kc: (b, 0, 0, 0)),
        ],
        out_shape=[
            jax.ShapeDtypeStruct((B, SQ, H * D), jnp.float32),
            jax.ShapeDtypeStruct((B, H, SQ, 1), jnp.float32),
        ],
        scratch_shapes=[
            pltpu.VMEM((SQ, H * KC), jnp.float32),
        ],
    )(Q3, K3, V3)


def _exchange_sum(num, l):

    def body(num_ref, l_ref, nsum_ref, lsum_ref, rnum_ref, rl_ref,
             nsend, nrecv, lsend, lrecv):
        my_x = lax.axis_index("x")
        my_y = lax.axis_index("y")
        my_z = lax.axis_index("z")
        peer = (1 - my_x, my_y, my_z)

        barrier = pltpu.get_barrier_semaphore()
        pl.semaphore_signal(
            barrier, inc=1, device_id=peer,
            device_id_type=pl.DeviceIdType.MESH,
        )
        pl.semaphore_wait(barrier, 1)

        rdma_n = pltpu.make_async_remote_copy(
            src_ref=num_ref, dst_ref=rnum_ref,
            send_sem=nsend, recv_sem=nrecv,
            device_id=peer, device_id_type=pl.DeviceIdType.MESH,
        )
        rdma_l = pltpu.make_async_remote_copy(
            src_ref=l_ref, dst_ref=rl_ref,
            send_sem=lsend, recv_sem=lrecv,
            device_id=peer, device_id_type=pl.DeviceIdType.MESH,
        )
        rdma_n.start()
        rdma_l.start()
        rdma_n.wait()
        rdma_l.wait()

        nsum_ref[...] = num_ref[...] + rnum_ref[...]
        lsum_ref[...] = l_ref[...] + rl_ref[...]

    return pl.pallas_call(
        body,
        in_specs=[
            pl.BlockSpec(memory_space=pltpu.VMEM),
            pl.BlockSpec(memory_space=pltpu.VMEM),
        ],
        out_specs=[
            pl.BlockSpec(memory_space=pltpu.VMEM),
            pl.BlockSpec(memory_space=pltpu.VMEM),
        ],
        out_shape=[
            jax.ShapeDtypeStruct((B, SQ, H * D), jnp.float32),
            jax.ShapeDtypeStruct((B, H, SQ, 1), jnp.float32),
        ],
        scratch_shapes=[
            pltpu.VMEM((B, SQ, H * D), jnp.float32),
            pltpu.VMEM((B, H, SQ, 1), jnp.float32),
            pltpu.SemaphoreType.DMA,
            pltpu.SemaphoreType.DMA,
            pltpu.SemaphoreType.DMA,
            pltpu.SemaphoreType.DMA,
        ],
        compiler_params=pltpu.CompilerParams(collective_id=0),
    )(num, l)


def _divide(nsum, lsum):

    def body(n_ref, l_ref, out_ref):
        out_ref[...] = n_ref[...] / l_ref[...]

    return pl.pallas_call(
        body,
        grid=(B, H),
        in_specs=[
            pl.BlockSpec((None, SQ, D), lambda b, h: (b, 0, h)),
            pl.BlockSpec((None, None, SQ, 1), lambda b, h: (b, h, 0, 0)),
        ],
        out_specs=pl.BlockSpec((None, SQ, D), lambda b, h: (b, 0, h)),
        out_shape=jax.ShapeDtypeStruct((B, SQ, H * D), jnp.float32),
    )(nsum, lsum)


def kernel(Q, K, V):
    skv = K.shape[1]
    Q3 = Q.reshape(B, SQ, H * D)
    K3 = K.reshape(B, skv, H * D)
    V3 = V.reshape(B, skv, H * D)
    num, l = _flash_partial(Q3, K3, V3, skv)
    nsum, lsum = _exchange_sum(num, l)
    out = _divide(nsum, lsum)
    return out.reshape(B, SQ, H, D)
